# baseline (speedup 1.0000x reference)
import numpy as np
import jax
import jax.numpy as jnp
from jax import lax
from jax.experimental import pallas as pl
from jax.experimental.pallas import tpu as pltpu
from jax.experimental.pallas import tpu_sc as plsc

VOCAB = 100000
D = 768
SEQ = 4096
BATCH = 4
BFLAT = BATCH * SEQ

NC, NS = 2, 16
NW = NC * NS
PPW = SEQ // NW
Q = 16
NQ = PPW // Q
T = BATCH * NQ
NBUF = 4
LANES = 16


def _pos_encoding() -> np.ndarray:
    pos = np.arange(SEQ, dtype=np.float64)[:, None]
    i2 = np.arange(0, D, 2, dtype=np.float64)
    enc = np.zeros((SEQ, D), dtype=np.float32)
    enc[:, 0::2] = np.sin(pos / 10000 ** (i2 / D)).astype(np.float32)
    enc[:, 1::2] = np.cos(pos / 10000 ** (i2 / D)).astype(np.float32)
    return enc


_POS = _pos_encoding()


def _body(x_hbm, pos_hbm, emb_hbm, out_hbm,
          idx_v, pos_v, rows_v, ps0, ps1, g0, g1, g2, g3, o0, o1, o2, o3):
    wid = lax.axis_index("s") * NC + lax.axis_index("c")
    p0 = wid * PPW

    ps = (ps0, ps1)
    gs = (g0, g1, g2, g3)
    os_ = (o0, o1, o2, o3)

    def gather_src(t):
        q, b = t // BATCH, t % BATCH
        return emb_hbm.at[idx_v.at[b, pl.ds(q * Q, Q)]]

    def out_dst(t):
        q, b = t // BATCH, t % BATCH
        return out_hbm.at[pl.ds(b * SEQ + p0 + q * Q, Q)]

    def pos_src(q):
        return pos_hbm.at[pl.ds(p0 + q * Q, Q)]

    pltpu.async_copy(pos_src(0), pos_v.at[0], ps[0])
    idx_cp = pltpu.async_copy(x_hbm.at[:, pl.ds(p0, PPW)], idx_v, ps[1])
    idx_cp.wait()
    for t in range(NBUF):
        pltpu.async_copy(gather_src(t), rows_v.at[t], gs[t])

    def chunk_step(qg, tq):
        q = qg + tq // BATCH
        b = tq % BATCH
        t = qg * BATCH + tq
        bi = tq % NBUF
        pq = (tq // BATCH) % 2

        def g_src(tt, qq):
            return emb_hbm.at[idx_v.at[b, pl.ds(qq * Q, Q)]]

        if b == 0:
            pltpu.make_async_copy(pos_src(q), pos_v.at[pq], ps[pq]).wait()

            @pl.when(q + 1 < NQ)
            def _():
                pltpu.async_copy(pos_src(q + 1), pos_v.at[1 - pq], ps[1 - pq])

        nbi = (tq + 1) % NBUF
        tn = t + 1 - NBUF
        bn = (tq + 1 - NBUF) % BATCH
        qn = qg + (tq + 1 - NBUF) // BATCH

        @pl.when(jnp.logical_and(tn >= 0, t + 1 < T))
        def _():
            pltpu.make_async_copy(
                rows_v.at[nbi],
                out_hbm.at[pl.ds(bn * SEQ + p0 + qn * Q, Q)],
                os_[nbi]).wait()
            qx = qg + (tq + 1) // BATCH
            bx = (tq + 1) % BATCH
            pltpu.async_copy(
                emb_hbm.at[idx_v.at[bx, pl.ds(qx * Q, Q)]],
                rows_v.at[nbi], gs[nbi])

        pltpu.make_async_copy(g_src(t, q), rows_v.at[bi], gs[bi]).wait()

        def row(r, _):
            for j in range(D // LANES):
                v = pos_v[pq, r, pl.ds(j * LANES, LANES)]
                plsc.addupdate(rows_v.at[bi, r, pl.ds(j * LANES, LANES)], v)
            return 0

        lax.fori_loop(0, Q, row, 0, unroll=2)
        pltpu.async_copy(rows_v.at[bi],
                         out_hbm.at[pl.ds(b * SEQ + p0 + q * Q, Q)], os_[bi])

    GRP = 2

    def group(i, _):
        for tq in range(GRP * BATCH):
            chunk_step(i * GRP, tq)
        return 0

    lax.fori_loop(0, NQ // GRP, group, 0, unroll=False)

    for t in range(T - NBUF, T):
        q, b = t // BATCH, t % BATCH
        pltpu.make_async_copy(rows_v.at[t % NBUF],
                              out_hbm.at[pl.ds(b * SEQ + p0 + q * Q, Q)],
                              os_[t % NBUF]).wait()


@jax.jit
def _run(x2, emb):
    mesh = plsc.VectorSubcoreMesh(core_axis_name="c", subcore_axis_name="s",
                                  num_cores=NC, num_subcores=NS)
    pos = jnp.asarray(_POS)
    return pl.kernel(
        _body,
        out_type=jax.ShapeDtypeStruct((BFLAT, D), jnp.float32),
        mesh=mesh,
        scratch_types=[
            pltpu.VMEM((BATCH, PPW), jnp.int32),
            pltpu.VMEM((2, Q, D), jnp.float32),
            pltpu.VMEM((NBUF, Q, D), jnp.float32),
        ] + [pltpu.SemaphoreType.DMA] * 10,
    )(x2, pos, emb)


def kernel(x, emb):
    out = _run(x.astype(jnp.int32), emb)
    return out.reshape(BATCH, SEQ, D)
